# baseline (speedup 1.0000x reference)
import dataclasses
import functools

import numpy as np
import jax
import jax.numpy as jnp
from jax.experimental import pallas as pl
from jax.experimental.pallas import tpu as pltpu
from jax.experimental.pallas import tpu_sc as plsc

TOKENS = 8192
HIDDEN = 2048
EXPERTS = 64

R_SC = 2048
BM = 1024
SC_BR = 8

_p = np.arange(HIDDEN)
_w = _p % 32
_PERM = (_p - _w) + (_w // 2) + 16 * (_w % 2)


def _sc_convert(x):
    mesh = plsc.VectorSubcoreMesh(core_axis_name="c", subcore_axis_name="s")
    cp = pltpu.CompilerParams()
    if "needs_layout_passes" in pltpu.CompilerParams.__dataclass_fields__:
        cp = dataclasses.replace(cp, needs_layout_passes=False)

    @functools.partial(
        pl.kernel,
        out_type=jax.ShapeDtypeStruct((R_SC, HIDDEN), jnp.bfloat16),
        mesh=mesh,
        compiler_params=cp,
    )
    def conv(x_hbm, o_hbm):
        def body(in_v, out_v):
            @pl.loop(0, SC_BR)
            def _(r):
                @pl.loop(0, HIDDEN, step=32)
                def _(c):
                    a = in_v[r, pl.ds(c, 16)]
                    b = in_v[r, pl.ds(c + 16, 16)]
                    out_v[r, pl.ds(c, 32)] = plsc.pack(
                        a, b, format=plsc.PackFormat.INTERLEAVED)

        pltpu.emit_pipeline(
            body,
            grid=(R_SC // SC_BR,),
            in_specs=[pl.BlockSpec((SC_BR, HIDDEN), lambda i: (i, 0))],
            out_specs=[pl.BlockSpec((SC_BR, HIDDEN), lambda i: (i, 0))],
            core_axis_name=("c", "s"),
            dimension_semantics=(pltpu.PARALLEL,),
        )(x_hbm, o_hbm)

    return conv(x)


def _mm_body(x_ref, w_ref, o_ref):
    x = x_ref[...]
    if x.dtype != jnp.bfloat16:
        x = x.astype(jnp.bfloat16)
    w = w_ref[...].astype(jnp.bfloat16)
    o_ref[...] = jax.lax.dot_general(
        x, w, (((1,), (1,)), ((), ())),
        preferred_element_type=jnp.float32)


def _tc_matmul(x, w, row_start, rows):
    grid = (rows // BM,)
    off = row_start // BM
    return pl.pallas_call(
        _mm_body,
        grid=grid,
        in_specs=[
            pl.BlockSpec((BM, HIDDEN), lambda i: (i + off, 0)),
            pl.BlockSpec((EXPERTS, HIDDEN), lambda i: (0, 0)),
        ],
        out_specs=pl.BlockSpec((BM, EXPERTS), lambda i: (i, 0)),
        out_shape=jax.ShapeDtypeStruct((rows, EXPERTS), jnp.float32),
    )(x, w)


@functools.partial(jax.jit, static_argnames=())
def kernel(x, W):
    xb = _sc_convert(x)
    out_hi = _tc_matmul(x, W, R_SC, TOKENS - R_SC)
    w_perm = W[:, _PERM]
    out_lo = _tc_matmul(xb, w_perm, 0, R_SC)
    return jnp.concatenate([out_lo, out_hi], axis=0)


# speedup vs baseline: 1.1013x; 1.1013x over previous
import dataclasses
import functools

import numpy as np
import jax
import jax.numpy as jnp
from jax.experimental import pallas as pl
from jax.experimental.pallas import tpu as pltpu
from jax.experimental.pallas import tpu_sc as plsc

TOKENS = 8192
HIDDEN = 2048
EXPERTS = 64

R_SC = 2048
BM = 1024
SC_BR = 16


def _sc_convert(x):
    mesh = plsc.VectorSubcoreMesh(core_axis_name="c", subcore_axis_name="s")
    cp = pltpu.CompilerParams()
    if "needs_layout_passes" in pltpu.CompilerParams.__dataclass_fields__:
        cp = dataclasses.replace(cp, needs_layout_passes=False)

    @functools.partial(
        pl.kernel,
        out_type=jax.ShapeDtypeStruct((R_SC, HIDDEN), jnp.bfloat16),
        mesh=mesh,
        compiler_params=cp,
    )
    def conv(x_hbm, o_hbm):
        def body(in_v, out_v):
            idx2 = jax.lax.iota(jnp.int32, 16) * 2

            @pl.loop(0, SC_BR)
            def _(r):
                row = jnp.full((16,), r, jnp.int32)
                for c in range(0, HIDDEN, 32):
                    idx_e = idx2 + c
                    a = plsc.load_gather(in_v, [row, idx_e])
                    b = plsc.load_gather(in_v, [row, idx_e + 1])
                    out_v[r, pl.ds(c, 32)] = plsc.pack(
                        a, b, format=plsc.PackFormat.INTERLEAVED)

        pltpu.emit_pipeline(
            body,
            grid=(R_SC // SC_BR,),
            in_specs=[pl.BlockSpec((SC_BR, HIDDEN), lambda i: (i, 0))],
            out_specs=[pl.BlockSpec((SC_BR, HIDDEN), lambda i: (i, 0))],
            core_axis_name=("c", "s"),
            dimension_semantics=(pltpu.PARALLEL,),
        )(x_hbm, o_hbm)

    return conv(x)


def _mm_body(x_ref, w_ref, o_ref):
    x = x_ref[...]
    if x.dtype != jnp.bfloat16:
        x = x.astype(jnp.bfloat16)
    w = w_ref[...].astype(jnp.bfloat16)
    o_ref[...] = jax.lax.dot_general(
        x, w, (((1,), (1,)), ((), ())),
        preferred_element_type=jnp.float32)


def _tc_matmul(x, w, row_start, rows):
    grid = (rows // BM,)
    off = row_start // BM
    return pl.pallas_call(
        _mm_body,
        grid=grid,
        in_specs=[
            pl.BlockSpec((BM, HIDDEN), lambda i: (i + off, 0)),
            pl.BlockSpec((EXPERTS, HIDDEN), lambda i: (0, 0)),
        ],
        out_specs=pl.BlockSpec((BM, EXPERTS), lambda i: (i, 0)),
        out_shape=jax.ShapeDtypeStruct((rows, EXPERTS), jnp.float32),
    )(x, w)


@functools.partial(jax.jit, static_argnames=())
def kernel(x, W):
    xb = _sc_convert(x)
    out_hi = _tc_matmul(x, W, R_SC, TOKENS - R_SC)
    out_lo = _tc_matmul(xb, W, 0, R_SC)
    return jnp.concatenate([out_lo, out_hi], axis=0)


# speedup vs baseline: 1.1394x; 1.0346x over previous
import dataclasses
import functools

import numpy as np
import jax
import jax.numpy as jnp
from jax.experimental import pallas as pl
from jax.experimental.pallas import tpu as pltpu
from jax.experimental.pallas import tpu_sc as plsc

TOKENS = 8192
HIDDEN = 2048
EXPERTS = 64

R_SC = 2048
BM = 1024
SC_BR = 16


def _sc_convert(x):
    mesh = plsc.VectorSubcoreMesh(core_axis_name="c", subcore_axis_name="s")
    cp = pltpu.CompilerParams()
    if "needs_layout_passes" in pltpu.CompilerParams.__dataclass_fields__:
        cp = dataclasses.replace(cp, needs_layout_passes=False)

    @functools.partial(
        pl.kernel,
        out_type=jax.ShapeDtypeStruct((R_SC, HIDDEN), jnp.bfloat16),
        mesh=mesh,
        compiler_params=cp,
    )
    def conv(x_hbm, o_hbm):
        def body(in_v, out_v):
            @pl.loop(0, SC_BR)
            def _(r):
                for c in range(0, HIDDEN, 32):
                    a = in_v[r, pl.ds(c, 16)]
                    b = in_v[r, pl.ds(c + 16, 16)]
                    out_v[r, pl.ds(c, 32)] = plsc.pack(
                        a, b, format=plsc.PackFormat.INTERLEAVED)

        pltpu.emit_pipeline(
            body,
            grid=(R_SC // SC_BR,),
            in_specs=[pl.BlockSpec((SC_BR, HIDDEN), lambda i: (i, 0))],
            out_specs=[pl.BlockSpec((SC_BR, HIDDEN), lambda i: (i, 0))],
            core_axis_name=("c", "s"),
            dimension_semantics=(pltpu.PARALLEL,),
        )(x_hbm, o_hbm)

    return conv(x)


def _mm_body(x_ref, w_ref, o_ref):
    x = x_ref[...]
    if x.dtype != jnp.bfloat16:
        x = x.astype(jnp.bfloat16)
    w = w_ref[...].astype(jnp.bfloat16)
    o_ref[...] = jax.lax.dot_general(
        x, w, (((1,), (1,)), ((), ())),
        preferred_element_type=jnp.float32)


def _tc_matmul(x, w, row_start, rows):
    grid = (rows // BM,)
    off = row_start // BM
    return pl.pallas_call(
        _mm_body,
        grid=grid,
        in_specs=[
            pl.BlockSpec((BM, HIDDEN), lambda i: (i + off, 0)),
            pl.BlockSpec((EXPERTS, HIDDEN), lambda i: (0, 0)),
        ],
        out_specs=pl.BlockSpec((BM, EXPERTS), lambda i: (i, 0)),
        out_shape=jax.ShapeDtypeStruct((rows, EXPERTS), jnp.float32),
    )(x, w)


@functools.partial(jax.jit, static_argnames=())
def kernel(x, W):
    xb = _sc_convert(x)
    out_hi = _tc_matmul(x, W, R_SC, TOKENS - R_SC)
    w_perm = (W.reshape(EXPERTS, HIDDEN // 32, 2, 16)
              .transpose(0, 1, 3, 2).reshape(EXPERTS, HIDDEN))
    out_lo = _tc_matmul(xb, w_perm, 0, R_SC)
    return jnp.concatenate([out_lo, out_hi], axis=0)
